# baseline (speedup 1.0000x reference)
import functools

import jax
import jax.numpy as jnp
from jax import lax
from jax.experimental import pallas as pl
from jax.experimental.pallas import tpu as pltpu
from jax.experimental.pallas import tpu_sc as plsc

_DIM = 64
_MAX_LEN = 8192
_LANES = 16
_NUM_WORKERS = 32
_CW = 128
_K = 8
_NBUF = 2
_SCH = _K * _CW
_SBLK = 512
_NCHUNK = 2


def _sc_gather_rows(position, embedding16):
    n = position.size
    per_worker = n // _NUM_WORKERS
    n_super = per_worker // _SCH

    idx2d = position.reshape(n // _CW, _CW).astype(jnp.int32)

    mesh = plsc.VectorSubcoreMesh(core_axis_name="c", subcore_axis_name="s")

    @functools.partial(
        pl.kernel,
        mesh=mesh,
        out_type=jax.ShapeDtypeStruct((n, _DIM), jnp.bfloat16),
        compiler_params=pltpu.CompilerParams(use_tc_tiling_on_sc=False),
        scratch_types=[
            pltpu.VMEM((_NBUF, _K, _CW), jnp.int32),
            pltpu.VMEM((_NBUF, _SCH, _DIM), jnp.bfloat16),
            pltpu.SemaphoreType.DMA,
            pltpu.SemaphoreType.DMA,
            pltpu.SemaphoreType.DMA,
            pltpu.SemaphoreType.DMA,
            pltpu.SemaphoreType.DMA,
            pltpu.SemaphoreType.DMA,
        ],
    )
    def sc_gather(emb_hbm, idx_hbm, out_hbm, idx_v, rows_v,
                  isem0, isem1, gsem0, gsem1, wsem0, wsem1):
        isem = (isem0, isem1)
        gsem = (gsem0, gsem1)
        wsem = (wsem0, wsem1)
        wid = lax.axis_index("s") * 2 + lax.axis_index("c")
        chunk_base = wid * (per_worker // _CW)
        row_base = wid * per_worker

        def fire_gathers(bb):
            for j in range(_K):
                pltpu.async_copy(
                    emb_hbm.at[idx_v.at[bb, j]],
                    rows_v.at[bb, pl.ds(j * _CW, _CW)], gsem[bb])

        def drain_gathers(bb):
            for j in range(_K):
                pltpu.make_async_copy(
                    emb_hbm.at[idx_v.at[bb, j]],
                    rows_v.at[bb, pl.ds(j * _CW, _CW)], gsem[bb]).wait()

        pltpu.async_copy(idx_hbm.at[pl.ds(chunk_base, _K)], idx_v.at[0],
                         isem[0])

        @pl.loop(0, n_super, step=_NBUF)
        def _(sc0):
            for bb in range(_NBUF):
                ob = 1 - bb
                sidx = sc0 + bb
                c0 = chunk_base + sidx * _K
                r0 = row_base + sidx * _SCH

                pltpu.make_async_copy(
                    idx_hbm.at[pl.ds(c0, _K)], idx_v.at[bb], isem[bb]).wait()
                for j in range(_K):
                    @pl.loop(0, _CW, step=_LANES)
                    def _(i):
                        v = idx_v[bb, j, pl.ds(i, _LANES)] + (_MAX_LEN - 1)
                        idx_v[bb, j, pl.ds(i, _LANES)] = jnp.clip(
                            v, 0, 2 * _MAX_LEN - 2)

                @pl.when(sidx >= _NBUF)
                def _():
                    pltpu.make_async_copy(
                        rows_v.at[bb],
                        out_hbm.at[pl.ds(r0 - _NBUF * _SCH, _SCH)],
                        wsem[bb]).wait()

                fire_gathers(bb)

                @pl.when(sidx >= 1)
                def _():
                    drain_gathers(ob)
                    pltpu.async_copy(
                        rows_v.at[ob], out_hbm.at[pl.ds(r0 - _SCH, _SCH)],
                        wsem[ob])

                @pl.when(sidx + 1 < n_super)
                def _():
                    pltpu.async_copy(
                        idx_hbm.at[pl.ds(c0 + _K, _K)], idx_v.at[ob],
                        isem[ob])

        last = n_super - 1
        drain_gathers(1)
        pltpu.async_copy(
            rows_v.at[1], out_hbm.at[pl.ds(row_base + last * _SCH, _SCH)],
            wsem[1])
        for bb in range(_NBUF):
            pltpu.make_async_copy(
                rows_v.at[bb], out_hbm.at[pl.ds(row_base, _SCH)],
                wsem[bb]).wait()

    return sc_gather(embedding16, idx2d)


def _tc_transpose_into(rows128, acc, boff, bsz, seq):
    bg = rows128.shape[0]
    half = _SBLK // 2

    def body(x_ref, *refs):
        o_ref = refs[-1]
        for k in range(seq // _SBLK):
            x_f32 = x_ref[0, pl.ds(k * half, half), :].astype(jnp.float32)
            x_t = x_f32.T
            o_ref[0, :, pl.ds(k * _SBLK, half)] = x_t[:_DIM, :]
            o_ref[0, :, pl.ds(k * _SBLK + half, half)] = x_t[_DIM:, :]

    in_specs = [pl.BlockSpec((1, seq // 2, 2 * _DIM), lambda i: (i, 0, 0))]
    inputs = [rows128]
    io_alias = {}
    if acc is not None:
        in_specs.append(pl.BlockSpec(memory_space=pl.ANY))
        inputs.append(acc)
        io_alias = {1: 0}
    return pl.pallas_call(
        body,
        grid=(bg,),
        in_specs=in_specs,
        out_specs=pl.BlockSpec((1, _DIM, seq), lambda i: (i + boff, 0, 0)),
        out_shape=jax.ShapeDtypeStruct((bsz, _DIM, seq), jnp.float32),
        input_output_aliases=io_alias,
        compiler_params=pltpu.CompilerParams(
            dimension_semantics=("parallel",)),
    )(*inputs)


def kernel(position, embedding):
    b, s = position.shape
    bg = b // _NCHUNK
    half = _SBLK // 2
    embedding16 = embedding.astype(jnp.bfloat16)
    acc = None
    for g in range(_NCHUNK):
        pos_g = position[g * bg:(g + 1) * bg]
        pos_perm = pos_g.reshape(bg, s // _SBLK, 2, half).swapaxes(2, 3)
        rows = _sc_gather_rows(pos_perm, embedding16)
        acc = _tc_transpose_into(
            rows.reshape(bg, s // 2, 2 * _DIM), acc, g * bg, b, s)
    return jnp.swapaxes(acc, 1, 2)
